# baseline (speedup 1.0000x reference)
import functools

import jax
import jax.numpy as jnp
from jax import lax
from jax.experimental import pallas as pl
from jax.experimental.pallas import tpu as pltpu
from jax.experimental.pallas import tpu_sc as plsc

_H = 256
_E = 8
_K = 384
_T = 8192
_TILE_A = 512
_TILE_D = 256
_NW = 32
_CHUNK = _T // _NW


def _router_kernel(x_ref, wg_ref, xa_ref, idx_ref, rank_ref, cnt_ref,
                   offs_ref, carry_ref):
    i = pl.program_id(0)

    @pl.when(i == 0)
    def _():
        carry_ref[...] = jnp.zeros_like(carry_ref)

    x = x_ref[...]
    logits = jnp.dot(x, wg_ref[...], preferred_element_type=jnp.float32)
    m = jnp.max(logits, axis=-1, keepdims=True)
    denom = jnp.sum(jnp.exp(logits - m), axis=-1, keepdims=True)
    gate = 1.0 / denom
    idx = jnp.argmax(logits, axis=-1)
    onehot = (jax.lax.broadcasted_iota(jnp.int32, (_TILE_A, _E), 1)
              == idx[:, None]).astype(jnp.float32)

    r_iota = jax.lax.broadcasted_iota(jnp.int32, (_TILE_A, _TILE_A), 0)
    c_iota = jax.lax.broadcasted_iota(jnp.int32, (_TILE_A, _TILE_A), 1)
    ltri = (c_iota < r_iota).astype(jnp.float32)
    excl = jnp.dot(ltri, onehot, preferred_element_type=jnp.float32)
    carry = carry_ref[0:1, 0:_E]
    rank_f = jnp.sum((excl + carry) * onehot, axis=-1)
    new_carry = carry + jnp.sum(onehot, axis=0, keepdims=True)

    carry_ref[0:1, 0:_E] = new_carry
    cnt16 = jnp.concatenate(
        [new_carry, jnp.zeros((1, _E), jnp.float32)], axis=1)
    cnt_i = cnt16.astype(jnp.int32)
    cnt_ref[...] = cnt_i
    p = jnp.concatenate([jnp.zeros((1, 1), jnp.int32), cnt_i[:, :15]], axis=1)
    for sh in (1, 2, 4, 8):
        p = p + jnp.concatenate(
            [jnp.zeros((1, sh), jnp.int32), p[:, :16 - sh]], axis=1)
    offs_ref[...] = p

    xg = x * gate
    gpad = jnp.where(
        jax.lax.broadcasted_iota(jnp.int32, (_TILE_A, _K - _H), 1) == 0,
        gate, 0.0)
    xa_ref[...] = jnp.concatenate([xg, gpad], axis=1)
    idx_ref[...] = idx.reshape(1, 1, _TILE_A)
    rank_ref[...] = rank_f.astype(jnp.int32).reshape(1, 1, _TILE_A)


def _run_router(xt, Wg):
    return pl.pallas_call(
        _router_kernel,
        grid=(_T // _TILE_A,),
        in_specs=[
            pl.BlockSpec((_TILE_A, _H), lambda i: (i, 0)),
            pl.BlockSpec((_H, _E), lambda i: (0, 0)),
        ],
        out_specs=[
            pl.BlockSpec((_TILE_A, _K), lambda i: (i, 0)),
            pl.BlockSpec((1, 1, _TILE_A), lambda i: (i, 0, 0)),
            pl.BlockSpec((1, 1, _TILE_A), lambda i: (i, 0, 0)),
            pl.BlockSpec((1, 16), lambda i: (0, 0)),
            pl.BlockSpec((1, 16), lambda i: (0, 0)),
        ],
        out_shape=[
            jax.ShapeDtypeStruct((_T, _K), jnp.float32),
            jax.ShapeDtypeStruct((_T // _TILE_A, 1, _TILE_A), jnp.int32),
            jax.ShapeDtypeStruct((_T // _TILE_A, 1, _TILE_A), jnp.int32),
            jax.ShapeDtypeStruct((1, 16), jnp.int32),
            jax.ShapeDtypeStruct((1, 16), jnp.int32),
        ],
        scratch_shapes=[pltpu.VMEM((1, 16), jnp.float32)],
    )(xt, Wg)


def _wid():
    return lax.axis_index("s") * 2 + lax.axis_index("c")


def _b_body(offs_hbm, idx_hbm, rank_hbm, xa_hbm, pos_hbm, xs_hbm,
            offs_v, idx_v, rank_v, posw_v, rows_v, sem):
    w = _wid()
    base = w * _CHUNK
    pltpu.sync_copy(offs_hbm, offs_v)
    pltpu.sync_copy(idx_hbm.at[pl.ds(base, _CHUNK)], idx_v)
    pltpu.sync_copy(rank_hbm.at[pl.ds(base, _CHUNK)], rank_v)
    pltpu.sync_copy(xa_hbm.at[pl.ds(base, _CHUNK)], rows_v)
    for j in range(_CHUNK // 16):
        iv = jnp.clip(idx_v[pl.ds(j * 16, 16)], 0, 15)
        rv = rank_v[pl.ds(j * 16, 16)]
        off = plsc.load_gather(offs_v, [iv])
        posw_v[j // 8, pl.ds((j % 8) * 16, 16)] = jnp.clip(
            off + rv, 0, _T - 1)
    pltpu.sync_copy(posw_v, pos_hbm.at[pl.ds(2 * w, 2)])
    for b in range(2):
        pltpu.async_copy(rows_v.at[pl.ds(b * 128, 128)],
                         xs_hbm.at[posw_v.at[b]], sem).wait()


def _run_b(offs, idx_l, rank_l, xa):
    mesh = plsc.VectorSubcoreMesh(core_axis_name="c", subcore_axis_name="s")
    f = functools.partial(
        pl.kernel, mesh=mesh,
        out_type=[
            jax.ShapeDtypeStruct((_T // 128, 128), jnp.int32),
            jax.ShapeDtypeStruct((_T, _K), jnp.float32),
        ],
        scratch_types=[
            pltpu.VMEM((16,), jnp.int32),
            pltpu.VMEM((_CHUNK,), jnp.int32),
            pltpu.VMEM((_CHUNK,), jnp.int32),
            pltpu.VMEM((2, 128), jnp.int32),
            pltpu.VMEM((_CHUNK, _K), jnp.float32),
            pltpu.SemaphoreType.DMA,
        ],
        compiler_params=pltpu.CompilerParams(needs_layout_passes=False),
    )(_b_body)
    return f(offs, idx_l, rank_l, xa)


def _gmm_kernel(cnt_ref, xs_ref, w_ref, o_ref):
    i = pl.program_id(0)
    row0 = i * _TILE_D
    x = xs_ref[...].astype(jnp.bfloat16)
    riota = jax.lax.broadcasted_iota(jnp.int32, (_TILE_D, 1), 0) + row0
    o_ref[...] = jnp.zeros((_TILE_D, _H), jnp.float32)
    lo = jnp.int32(0)
    for e in range(_E):
        hi = lo + cnt_ref[e]

        @pl.when(jnp.logical_and(hi > row0, lo < row0 + _TILE_D))
        def _(lo=lo, hi=hi):
            part = jnp.dot(x, w_ref[e], preferred_element_type=jnp.float32)
            mask = jnp.logical_and(riota >= lo, riota < hi)
            o_ref[...] += jnp.where(mask, part, 0.0)

        lo = hi


def _run_gmm(counts, xs, Waug):
    return pl.pallas_call(
        _gmm_kernel,
        grid_spec=pltpu.PrefetchScalarGridSpec(
            num_scalar_prefetch=1,
            grid=(_T // _TILE_D,),
            in_specs=[
                pl.BlockSpec((_TILE_D, _K), lambda i, c: (i, 0)),
                pl.BlockSpec((_E, _K, _H), lambda i, c: (0, 0, 0)),
            ],
            out_specs=pl.BlockSpec((_TILE_D, _H), lambda i, c: (i, 0)),
        ),
        out_shape=jax.ShapeDtypeStruct((_T, _H), jnp.float32),
    )(counts, xs, Waug)


def _c_body(ys_hbm, pos_hbm, out_hbm, posw_v, rows_v, sem):
    w = _wid()
    base = w * _CHUNK
    pltpu.sync_copy(pos_hbm.at[pl.ds(2 * w, 2)], posw_v)
    for b in range(2):
        pltpu.async_copy(ys_hbm.at[posw_v.at[b]],
                         rows_v.at[pl.ds(b * 128, 128)], sem).wait()
    pltpu.sync_copy(rows_v, out_hbm.at[pl.ds(base, _CHUNK)])


def _run_c(ys, pos):
    mesh = plsc.VectorSubcoreMesh(core_axis_name="c", subcore_axis_name="s")
    f = functools.partial(
        pl.kernel, mesh=mesh,
        out_type=jax.ShapeDtypeStruct((_T, _H), jnp.float32),
        scratch_types=[
            pltpu.VMEM((2, 128), jnp.int32),
            pltpu.VMEM((_CHUNK, _H), jnp.float32),
            pltpu.SemaphoreType.DMA,
        ],
    )(_c_body)
    return f(ys, pos)


def kernel(x, Wg, We, be):
    B, S, H = x.shape
    xt = x.reshape(-1, H)

    xa, idx3, rank3, cnt2, offs2 = _run_router(xt, Wg)
    idx_l = idx3.reshape(_T)
    rank_l = rank3.reshape(_T)
    counts = cnt2.reshape(16)
    offs = offs2.reshape(16)

    pos2d, xs = _run_b(offs, idx_l, rank_l, xa)

    Waug = jnp.concatenate(
        [We, be[:, None, :],
         jnp.zeros((_E, _K - _H - 1, _H), jnp.float32)],
        axis=1).astype(jnp.bfloat16)
    ys = _run_gmm(counts[:_E], xs, Waug)

    out = _run_c(ys, pos2d)
    return out.reshape(B, S, H)


# speedup vs baseline: 2.8243x; 2.8243x over previous
import jax
import jax.numpy as jnp
from jax.experimental import pallas as pl

_H = 256
_E = 8
_TILE = 512


def _moe_dense_kernel(x_ref, wg_ref, ws_ref, be_ref, o_ref):
    x = x_ref[...]
    logits = jnp.dot(x, wg_ref[...], preferred_element_type=jnp.float32)
    m = jnp.max(logits, axis=-1, keepdims=True)
    denom = jnp.sum(jnp.exp(logits - m), axis=-1, keepdims=True)
    gate = 1.0 / denom
    idx = jnp.argmax(logits, axis=-1)
    onehot = (jax.lax.broadcasted_iota(jnp.int32, (_TILE, _E), 1)
              == idx[:, None])
    og = jnp.where(onehot, gate, 0.0)
    x8 = jnp.concatenate(
        [(x * og[:, e:e + 1]).astype(jnp.bfloat16) for e in range(_E)],
        axis=1)
    acc = jnp.dot(og, be_ref[...], preferred_element_type=jnp.float32)
    acc = acc + jnp.dot(x8, ws_ref[...], preferred_element_type=jnp.float32)
    o_ref[...] = acc


def kernel(x, Wg, We, be):
    B, S, H = x.shape
    xt = x.reshape(-1, H)
    T = xt.shape[0]
    Ws = We.reshape(_E * H, H).astype(jnp.bfloat16)
    out = pl.pallas_call(
        _moe_dense_kernel,
        grid=(T // _TILE,),
        in_specs=[
            pl.BlockSpec((_TILE, H), lambda i: (i, 0)),
            pl.BlockSpec((H, _E), lambda i: (0, 0)),
            pl.BlockSpec((_E * H, H), lambda i: (0, 0)),
            pl.BlockSpec((_E, H), lambda i: (0, 0)),
        ],
        out_specs=pl.BlockSpec((_TILE, H), lambda i: (i, 0)),
        out_shape=jax.ShapeDtypeStruct((T, H), jnp.float32),
    )(xt, Wg, Ws, be)
    return out.reshape(B, S, H)
